# Initial kernel scaffold; baseline (speedup 1.0000x reference)
import jax
import jax.numpy as jnp
from jax.experimental import pallas as pl


def kernel(position_matrix, channel_matrix, W1_0, b1_0, W1_1, b1_1, W_res, W2_0, b2_0, W2_1, b2_1, gamma1, beta1, gamma2, beta2, gamma_r, beta_r, n_select_0, n_select_1, n_select_2):
    raise NotImplementedError("write your pallas kernel here")


# speedup vs baseline: 10.1645x; 10.1645x over previous
import functools

import jax
import jax.numpy as jnp
from jax import lax
from jax.experimental import pallas as pl
from jax.experimental.pallas import tpu as pltpu
from jax.experimental.pallas import tpu_sc as plsc

_B = 2
_N = 8192
_K = 16
_CIN = 32
_COUT = 64
_EPS = 1e-5

_NC = 2
_NS = 16
_NW = _NC * _NS
_D = 128


def _topk_body(pos_ref, posT_ref, idx_ref, s_ref, *, npts, rc):
  b = pl.program_id(0)
  p = pos_ref[0]
  d2 = None
  for d in range(3):
    cd = posT_ref[0, d, :]
    diff = cd[None, :] - p[:, d:d + 1]
    sq = diff * diff
    d2 = sq if d2 is None else d2 + sq
  s_ref[...] = -d2
  rowid = lax.broadcasted_iota(jnp.int32, (npts, rc), 0)
  base = b * npts
  for kk in range(_K):
    sm = s_ref[...]
    m = jnp.max(sm, axis=0)
    amax = jnp.min(jnp.where(sm == m[None, :], rowid, npts), axis=0)
    idx_ref[0, kk, :] = amax + base
    s_ref[...] = jnp.where(rowid == amax[None, :], -jnp.inf, sm)


def _topk(pos, npts, nctr, rc=128):
  posT = jnp.swapaxes(pos, 1, 2)
  grid = (_B, nctr // rc)
  return pl.pallas_call(
      functools.partial(_topk_body, npts=npts, rc=rc),
      grid=grid,
      in_specs=[
          pl.BlockSpec((1, npts, 3), lambda b, i: (b, 0, 0)),
          pl.BlockSpec((1, 3, rc), lambda b, i: (b, 0, i)),
      ],
      out_specs=pl.BlockSpec((1, _K, rc), lambda b, i: (b, 0, i)),
      out_shape=jax.ShapeDtypeStruct((_B, _K, nctr), jnp.int32),
      scratch_shapes=[pltpu.VMEM((npts, rc), jnp.float32)],
  )(pos, posT)


def _a1_body(pos_ref, feat_ref, w1p_ref, w1f_ref, b1_ref, out_ref):
  p = pos_ref[0]
  f = feat_ref[0]
  acc = jnp.dot(f, w1f_ref[...], preferred_element_type=jnp.float32)
  for d in range(3):
    acc = acc + p[:, d:d + 1] * w1p_ref[d, :][None, :]
  acc = acc + b1_ref[...]
  out_ref[0] = jnp.concatenate(
      [acc, jnp.zeros((acc.shape[0], _D - _COUT), jnp.float32)], axis=1)


def _build_a1(pos, feat, w1p, w1f, b1, r=512):
  grid = (_B, _N // r)
  return pl.pallas_call(
      _a1_body,
      grid=grid,
      in_specs=[
          pl.BlockSpec((1, r, 3), lambda b, i: (b, i, 0)),
          pl.BlockSpec((1, r, _CIN), lambda b, i: (b, i, 0)),
          pl.BlockSpec((3, _COUT), lambda b, i: (0, 0)),
          pl.BlockSpec((_CIN, _COUT), lambda b, i: (0, 0)),
          pl.BlockSpec((1, _COUT), lambda b, i: (0, 0)),
      ],
      out_specs=pl.BlockSpec((1, r, _D), lambda b, i: (b, i, 0)),
      out_shape=jax.ShapeDtypeStruct((_B, _N, _D), jnp.float32),
  )(pos, feat, w1p, w1f, b1)


def _a2_body(pos_ref, ch_ref, sums_ref, w2p_ref, w2f_ref, b2_ref, g1_ref,
             be1_ref, out_ref, *, count):
  mean = sums_ref[0, :] / count
  var = sums_ref[1, :] / count - mean * mean
  a1 = g1_ref[0, :] * lax.rsqrt(var + _EPS)
  c1 = be1_ref[0, :] - mean * a1
  ch = ch_ref[0]
  chbn = ch * a1[None, :] + c1[None, :]
  acc = jnp.dot(chbn, w2f_ref[...], preferred_element_type=jnp.float32)
  p = pos_ref[0]
  for d in range(3):
    acc = acc + p[:, d:d + 1] * w2p_ref[d, :][None, :]
  acc = acc + b2_ref[...]
  out_ref[0] = jnp.concatenate(
      [acc, jnp.zeros((acc.shape[0], _D - _COUT), jnp.float32)], axis=1)


def _build_a2(pos1, ch1, sums1, w2p, w2f, b2, g1, be1, r=512):
  n1 = _N // 2
  grid = (_B, n1 // r)
  return pl.pallas_call(
      functools.partial(_a2_body, count=float(_B * n1)),
      grid=grid,
      in_specs=[
          pl.BlockSpec((1, r, 3), lambda b, i: (b, i, 0)),
          pl.BlockSpec((1, r, _COUT), lambda b, i: (b, i, 0)),
          pl.BlockSpec((2, _COUT), lambda b, i: (0, 0)),
          pl.BlockSpec((3, _COUT), lambda b, i: (0, 0)),
          pl.BlockSpec((_COUT, _COUT), lambda b, i: (0, 0)),
          pl.BlockSpec((1, _COUT), lambda b, i: (0, 0)),
          pl.BlockSpec((1, _COUT), lambda b, i: (0, 0)),
          pl.BlockSpec((1, _COUT), lambda b, i: (0, 0)),
      ],
      out_specs=pl.BlockSpec((1, r, _D), lambda b, i: (b, i, 0)),
      out_shape=jax.ShapeDtypeStruct((_B, n1, _D), jnp.float32),
  )(pos1, ch1, sums1, w2p, w2f, b2, g1, be1)


def _sc_gather(table, idx3):
  ch = idx3.shape[1]
  d = table.shape[1]
  mesh = plsc.VectorSubcoreMesh(core_axis_name="c", subcore_axis_name="s")

  @functools.partial(
      pl.kernel,
      mesh=mesh,
      out_type=jax.ShapeDtypeStruct((_NW * ch * 128, d), jnp.float32),
      scratch_types=[
          pltpu.VMEM((ch, 128), jnp.int32),
          pltpu.VMEM((128, d), jnp.float32),
          pltpu.VMEM((128, d), jnp.float32),
          pltpu.SemaphoreType.DMA,
          pltpu.SemaphoreType.DMA,
      ],
  )
  def k(table_hbm, idx_hbm, out_hbm, idx_v, buf0, buf1, sem0, sem1):
    wid = lax.axis_index("s") * _NC + lax.axis_index("c")
    base = wid * (ch * 128)
    pltpu.sync_copy(idx_hbm.at[wid], idx_v)
    bufs = (buf0, buf1)
    sems = (sem0, sem1)
    handles = [None, None]
    for j in range(ch):
      s = j % 2
      handles[s] = pltpu.async_copy(table_hbm.at[idx_v.at[j]], bufs[s], sems[s])
      if j > 0:
        ps = (j - 1) % 2
        handles[ps].wait()
        pltpu.sync_copy(bufs[ps], out_hbm.at[pl.ds(base + (j - 1) * 128, 128)])
    last = (ch - 1) % 2
    handles[last].wait()
    pltpu.sync_copy(bufs[last], out_hbm.at[pl.ds(base + (ch - 1) * 128, 128)])

  return k(table, idx3)


def _gather_rows(a_flat, idx):
  b, k, nc = idx.shape
  total = b * k * nc
  idx3 = idx.reshape(_NW, total // (_NW * 128), 128)
  out = _sc_gather(a_flat, idx3)
  return out.reshape(b, k, nc, _D)


def _mlp1_body(g_ref, pos_ref, w1p_ref, w_ref, b_ref, out_ref, sums_ref):
  p = pos_ref[0]
  c = jnp.zeros((p.shape[0], _COUT), jnp.float32)
  for d in range(3):
    c = c + p[:, d:d + 1] * w1p_ref[d, :][None, :]
  acc = None
  for kk in range(_K):
    h = jnp.maximum(g_ref[0, kk][:, :_COUT] - c, 0.0)
    y = jnp.dot(h, w_ref[...], preferred_element_type=jnp.float32) + b_ref[...]
    y = jnp.maximum(y, 0.0)
    acc = y if acc is None else jnp.maximum(acc, y)
  out_ref[0] = acc
  first = (pl.program_id(0) == 0) & (pl.program_id(1) == 0)

  @pl.when(first)
  def _():
    sums_ref[...] = jnp.zeros_like(sums_ref)

  ssum = jnp.sum(acc, axis=0, keepdims=True)
  ssq = jnp.sum(acc * acc, axis=0, keepdims=True)
  sums_ref[...] += jnp.concatenate([ssum, ssq], axis=0)


def _mlp1(g, pos, w1p, w11, b11, r=256):
  n1 = _N // 2
  grid = (_B, n1 // r)
  return pl.pallas_call(
      _mlp1_body,
      grid=grid,
      in_specs=[
          pl.BlockSpec((1, _K, r, _D), lambda b, i: (b, 0, i, 0)),
          pl.BlockSpec((1, r, 3), lambda b, i: (b, i, 0)),
          pl.BlockSpec((3, _COUT), lambda b, i: (0, 0)),
          pl.BlockSpec((_COUT, _COUT), lambda b, i: (0, 0)),
          pl.BlockSpec((1, _COUT), lambda b, i: (0, 0)),
      ],
      out_specs=[
          pl.BlockSpec((1, r, _COUT), lambda b, i: (b, i, 0)),
          pl.BlockSpec((2, _COUT), lambda b, i: (0, 0)),
      ],
      out_shape=[
          jax.ShapeDtypeStruct((_B, n1, _COUT), jnp.float32),
          jax.ShapeDtypeStruct((2, _COUT), jnp.float32),
      ],
  )(g, pos, w1p, w11, b11)


def _mlp2_body(g_ref, pos_ref, feat_ref, w2p_ref, w_ref, b_ref, wres_ref,
               ch_ref, res_ref, sums_ref):
  p = pos_ref[0]
  c = jnp.zeros((p.shape[0], _COUT), jnp.float32)
  for d in range(3):
    c = c + p[:, d:d + 1] * w2p_ref[d, :][None, :]
  acc = None
  for kk in range(_K):
    h = jnp.maximum(g_ref[0, kk][:, :_COUT] - c, 0.0)
    y = jnp.dot(h, w_ref[...], preferred_element_type=jnp.float32) + b_ref[...]
    y = jnp.maximum(y, 0.0)
    acc = y if acc is None else jnp.maximum(acc, y)
  ch_ref[0] = acc
  res = jnp.dot(feat_ref[0], wres_ref[...], preferred_element_type=jnp.float32)
  res_ref[0] = res
  first = (pl.program_id(0) == 0) & (pl.program_id(1) == 0)

  @pl.when(first)
  def _():
    sums_ref[...] = jnp.zeros_like(sums_ref)

  sums_ref[...] += jnp.concatenate([
      jnp.sum(acc, axis=0, keepdims=True),
      jnp.sum(acc * acc, axis=0, keepdims=True),
      jnp.sum(res, axis=0, keepdims=True),
      jnp.sum(res * res, axis=0, keepdims=True),
  ], axis=0)


def _mlp2(g, pos, feat, w2p, w21, b21, wres, r=256):
  n2 = _N // 4
  grid = (_B, n2 // r)
  return pl.pallas_call(
      _mlp2_body,
      grid=grid,
      in_specs=[
          pl.BlockSpec((1, _K, r, _D), lambda b, i: (b, 0, i, 0)),
          pl.BlockSpec((1, r, 3), lambda b, i: (b, i, 0)),
          pl.BlockSpec((1, r, _CIN), lambda b, i: (b, i, 0)),
          pl.BlockSpec((3, _COUT), lambda b, i: (0, 0)),
          pl.BlockSpec((_COUT, _COUT), lambda b, i: (0, 0)),
          pl.BlockSpec((1, _COUT), lambda b, i: (0, 0)),
          pl.BlockSpec((_CIN, _COUT), lambda b, i: (0, 0)),
      ],
      out_specs=[
          pl.BlockSpec((1, r, _COUT), lambda b, i: (b, i, 0)),
          pl.BlockSpec((1, r, _COUT), lambda b, i: (b, i, 0)),
          pl.BlockSpec((4, _COUT), lambda b, i: (0, 0)),
      ],
      out_shape=[
          jax.ShapeDtypeStruct((_B, n2, _COUT), jnp.float32),
          jax.ShapeDtypeStruct((_B, n2, _COUT), jnp.float32),
          jax.ShapeDtypeStruct((4, _COUT), jnp.float32),
      ],
  )(g, pos, feat, w2p, w21, b21, wres)


def _fin_body(ch_ref, res_ref, sums_ref, g2_ref, be2_ref, gr_ref, br_ref,
              out_ref, *, count):
  m2 = sums_ref[0, :] / count
  v2 = sums_ref[1, :] / count - m2 * m2
  a2 = g2_ref[0, :] * lax.rsqrt(v2 + _EPS)
  c2 = be2_ref[0, :] - m2 * a2
  mr = sums_ref[2, :] / count
  vr = sums_ref[3, :] / count - mr * mr
  ar = gr_ref[0, :] * lax.rsqrt(vr + _EPS)
  cr = br_ref[0, :] - mr * ar
  out_ref[0] = (ch_ref[0] * a2[None, :] + c2[None, :]
                + res_ref[0] * ar[None, :] + cr[None, :])


def _finalize(ch2, res2, sums, g2, be2, gr, br, r=512):
  n2 = _N // 4
  grid = (_B, n2 // r)
  return pl.pallas_call(
      functools.partial(_fin_body, count=float(_B * n2)),
      grid=grid,
      in_specs=[
          pl.BlockSpec((1, r, _COUT), lambda b, i: (b, i, 0)),
          pl.BlockSpec((1, r, _COUT), lambda b, i: (b, i, 0)),
          pl.BlockSpec((4, _COUT), lambda b, i: (0, 0)),
          pl.BlockSpec((1, _COUT), lambda b, i: (0, 0)),
          pl.BlockSpec((1, _COUT), lambda b, i: (0, 0)),
          pl.BlockSpec((1, _COUT), lambda b, i: (0, 0)),
          pl.BlockSpec((1, _COUT), lambda b, i: (0, 0)),
      ],
      out_specs=pl.BlockSpec((1, r, _COUT), lambda b, i: (b, i, 0)),
      out_shape=jax.ShapeDtypeStruct((_B, n2, _COUT), jnp.float32),
  )(ch2, res2, sums, g2, be2, gr, br)


def kernel(position_matrix, channel_matrix, W1_0, b1_0, W1_1, b1_1, W_res,
           W2_0, b2_0, W2_1, b2_1, gamma1, beta1, gamma2, beta2, gamma_r,
           beta_r, n_select_0, n_select_1, n_select_2):
  n0, n1, n2 = _N, _N // 2, _N // 4
  zero = ((jnp.asarray(n_select_0) - n0)
          + (jnp.asarray(n_select_1) - n1)
          + (jnp.asarray(n_select_2) - n2)).astype(position_matrix.dtype)

  pos = position_matrix
  feat = channel_matrix
  w1p, w1f = W1_0[:3], W1_0[3:]
  w2p, w2f = W2_0[:3], W2_0[3:]
  row = lambda v: v.reshape(1, _COUT)

  idx1 = _topk(pos, n0, n1)
  a1 = _build_a1(pos, feat, w1p, w1f, row(b1_0))
  g1 = _gather_rows(a1.reshape(_B * n0, _D), idx1)
  ch1, sums1 = _mlp1(g1, pos, w1p, W1_1, row(b1_1))

  pos1 = pos[:, :n1]
  idx2 = _topk(pos1, n1, n2)
  a2 = _build_a2(pos1, ch1, sums1, w2p, w2f, row(b2_0), row(gamma1),
                 row(beta1))
  g2 = _gather_rows(a2.reshape(_B * n1, _D), idx2)
  ch2_raw, res2, sums2 = _mlp2(g2, pos1, feat[:, :n2], w2p, W2_1, row(b2_1),
                               W_res)
  ch2 = _finalize(ch2_raw, res2, sums2, row(gamma2), row(beta2), row(gamma_r),
                  row(beta_r))
  pos2 = pos[:, :n2] + zero
  return (pos2, ch2)


# speedup vs baseline: 23.2366x; 2.2861x over previous
import functools

import jax
import jax.numpy as jnp
from jax import lax
from jax.experimental import pallas as pl
from jax.experimental.pallas import tpu as pltpu
from jax.experimental.pallas import tpu_sc as plsc

_B = 2
_N = 8192
_K = 16
_CIN = 32
_COUT = 64
_EPS = 1e-5

_NC = 2
_NS = 16
_NW = _NC * _NS
_D = 128


def _score_body(pos_ref, posT_ref, s_ref, seg_ref, *, npts, rs):
  d2 = None
  for d in range(3):
    cd = pos_ref[0][:, d:d + 1]
    pd = posT_ref[0, d, :][None, :]
    diff = cd - pd
    sq = diff * diff
    d2 = sq if d2 is None else d2 + sq
  s = -d2
  s_ref[0] = s
  nseg = npts // 128
  for g in range(nseg):
    seg_ref[0, :, g] = jnp.max(s[:, g * 128:(g + 1) * 128], axis=1)


def _scores(pos, npts, nctr, rs=128):
  posT = jnp.swapaxes(pos, 1, 2)
  nseg = npts // 128
  grid = (_B, nctr // rs)
  return pl.pallas_call(
      functools.partial(_score_body, npts=npts, rs=rs),
      grid=grid,
      in_specs=[
          pl.BlockSpec((1, rs, 3), lambda b, i: (b, i, 0)),
          pl.BlockSpec((1, 3, npts), lambda b, i: (b, 0, 0)),
      ],
      out_specs=[
          pl.BlockSpec((1, rs, npts), lambda b, i: (b, i, 0)),
          pl.BlockSpec((1, rs, nseg), lambda b, i: (b, i, 0)),
      ],
      out_shape=[
          jax.ShapeDtypeStruct((_B, nctr, npts), jnp.float32),
          jax.ShapeDtypeStruct((_B, nctr, nseg), jnp.float32),
      ],
  )(pos, posT)


def _sc_extract(scores2, seg2, npts, nctr):
  rt = _B * nctr
  nseg = npts // 128
  q = nseg // 16
  rows_per = rt // _NW
  nit = rows_per // 2
  sh = nctr.bit_length() - 1
  neg = jnp.float32(-3.0e38)
  bigi = jnp.int32(1 << 20)
  mesh = plsc.VectorSubcoreMesh(core_axis_name="c", subcore_axis_name="s")

  @functools.partial(
      pl.kernel,
      mesh=mesh,
      out_type=jax.ShapeDtypeStruct((rt * _K,), jnp.int32),
      scratch_types=[
          pltpu.VMEM((npts,), jnp.float32),
          pltpu.VMEM((npts,), jnp.float32),
          pltpu.VMEM((rows_per, nseg), jnp.float32),
          pltpu.VMEM((2 * _K,), jnp.int32),
          pltpu.SemaphoreType.DMA,
          pltpu.SemaphoreType.DMA,
      ],
  )
  def k(s_hbm, g_hbm, o_hbm, row0, row1, segall, idxst, sem0, sem1):
    wid = lax.axis_index("s") * _NC + lax.axis_index("c")
    base = wid * rows_per
    pltpu.sync_copy(g_hbm.at[pl.ds(base, rows_per)], segall)
    iota16 = lax.iota(jnp.int32, 16)
    dnums = lax.GatherDimensionNumbers(
        offset_dims=(), collapsed_slice_dims=(0,), start_index_map=(0,))

    def vperm(x, perm):
      return lax.gather(x, perm[:, None], dnums, slice_sizes=(1,),
                        mode=lax.GatherScatterMode.PROMISE_IN_BOUNDS)

    def bfly(x, op):
      for s in (8, 4, 2, 1):
        x = op(x, vperm(x, iota16 ^ s))
      return x

    def to_scalar(splat_i32):
      return splat_i32[0]

    def extract_row(row_ref, rloc, rglob):
      gbase = (rglob >> sh) * npts
      sv = [segall[rloc, pl.ds(qq * 16, 16)] for qq in range(q)]
      idxacc = jnp.zeros((16,), jnp.int32)
      for t in range(_K):
        m = sv[0]
        for qq in range(1, q):
          m = jnp.maximum(m, sv[qq])
        m = bfly(m, jnp.maximum)
        pk = None
        for qq in range(q):
          cand = jnp.where(sv[qq] == m, iota16 + qq * 16, bigi)
          pk = cand if pk is None else jnp.minimum(pk, cand)
        sstar = bfly(pk, jnp.minimum)
        col0 = to_scalar(sstar) * 128
        ch = [row_ref[pl.ds(col0 + v * 16, 16)] for v in range(8)]
        jp = None
        for v in range(8):
          cand = jnp.where(ch[v] == m, iota16 + v * 16, bigi)
          jp = cand if jp is None else jnp.minimum(jp, cand)
        jloc = bfly(jp, jnp.minimum)
        jloc_s = to_scalar(jloc)
        cb = col0 + (jloc_s >> 4) * 16
        cv = row_ref[pl.ds(cb, 16)]
        row_ref[pl.ds(cb, 16)] = jnp.where(iota16 == (jloc_s & 15), neg, cv)
        nm = None
        for v in range(8):
          cm = jnp.where(iota16 + v * 16 == jloc, neg, ch[v])
          nm = cm if nm is None else jnp.maximum(nm, cm)
        nms = bfly(nm, jnp.maximum)
        for qq in range(q):
          sv[qq] = jnp.where(iota16 + qq * 16 == sstar, nms, sv[qq])
        idxacc = jnp.where(iota16 == t,
                           gbase + col0 + (jloc & jnp.int32(127)), idxacc)
      return idxacc

    pltpu.sync_copy(s_hbm.at[base], row0)
    pltpu.async_copy(s_hbm.at[base + 1], row1, sem1)

    def body(i, _):
      r0 = base + 2 * i
      idxst[pl.ds(0, 16)] = extract_row(row0, 2 * i, r0)
      nxt0 = jnp.minimum(r0 + 2, rt - 1)
      pltpu.async_copy(s_hbm.at[nxt0], row0, sem0)
      pltpu.make_async_copy(s_hbm.at[base], row1, sem1).wait()
      idxst[pl.ds(16, 16)] = extract_row(row1, 2 * i + 1, r0 + 1)
      nxt1 = jnp.minimum(r0 + 3, rt - 1)
      pltpu.async_copy(s_hbm.at[nxt1], row1, sem1)
      pltpu.sync_copy(idxst, o_hbm.at[pl.ds(r0 * _K, 2 * _K)])
      pltpu.make_async_copy(s_hbm.at[base], row0, sem0).wait()
      return 0

    lax.fori_loop(0, nit, body, 0)
    pltpu.make_async_copy(s_hbm.at[base], row1, sem1).wait()

  return k(scores2, seg2)


def _topk(pos, npts, nctr):
  s, seg = _scores(pos, npts, nctr)
  idx = _sc_extract(s.reshape(_B * nctr, npts),
                    seg.reshape(_B * nctr, npts // 128), npts, nctr)
  return idx.reshape(_B, nctr, _K)


def _a1_body(pos_ref, feat_ref, w1p_ref, w1f_ref, b1_ref, out_ref):
  p = pos_ref[0]
  f = feat_ref[0]
  acc = jnp.dot(f, w1f_ref[...], preferred_element_type=jnp.float32)
  for d in range(3):
    acc = acc + p[:, d:d + 1] * w1p_ref[d, :][None, :]
  acc = acc + b1_ref[...]
  out_ref[0] = jnp.concatenate(
      [acc, jnp.zeros((acc.shape[0], _D - _COUT), jnp.float32)], axis=1)


def _build_a1(pos, feat, w1p, w1f, b1, r=512):
  grid = (_B, _N // r)
  return pl.pallas_call(
      _a1_body,
      grid=grid,
      in_specs=[
          pl.BlockSpec((1, r, 3), lambda b, i: (b, i, 0)),
          pl.BlockSpec((1, r, _CIN), lambda b, i: (b, i, 0)),
          pl.BlockSpec((3, _COUT), lambda b, i: (0, 0)),
          pl.BlockSpec((_CIN, _COUT), lambda b, i: (0, 0)),
          pl.BlockSpec((1, _COUT), lambda b, i: (0, 0)),
      ],
      out_specs=pl.BlockSpec((1, r, _D), lambda b, i: (b, i, 0)),
      out_shape=jax.ShapeDtypeStruct((_B, _N, _D), jnp.float32),
  )(pos, feat, w1p, w1f, b1)


def _a2_body(pos_ref, ch_ref, sums_ref, w2p_ref, w2f_ref, b2_ref, g1_ref,
             be1_ref, out_ref, *, count):
  mean = sums_ref[0, :] / count
  var = sums_ref[1, :] / count - mean * mean
  a1 = g1_ref[0, :] * lax.rsqrt(var + _EPS)
  c1 = be1_ref[0, :] - mean * a1
  ch = ch_ref[0]
  chbn = ch * a1[None, :] + c1[None, :]
  acc = jnp.dot(chbn, w2f_ref[...], preferred_element_type=jnp.float32)
  p = pos_ref[0]
  for d in range(3):
    acc = acc + p[:, d:d + 1] * w2p_ref[d, :][None, :]
  acc = acc + b2_ref[...]
  out_ref[0] = jnp.concatenate(
      [acc, jnp.zeros((acc.shape[0], _D - _COUT), jnp.float32)], axis=1)


def _build_a2(pos1, ch1, sums1, w2p, w2f, b2, g1, be1, r=512):
  n1 = _N // 2
  grid = (_B, n1 // r)
  return pl.pallas_call(
      functools.partial(_a2_body, count=float(_B * n1)),
      grid=grid,
      in_specs=[
          pl.BlockSpec((1, r, 3), lambda b, i: (b, i, 0)),
          pl.BlockSpec((1, r, _COUT), lambda b, i: (b, i, 0)),
          pl.BlockSpec((2, _COUT), lambda b, i: (0, 0)),
          pl.BlockSpec((3, _COUT), lambda b, i: (0, 0)),
          pl.BlockSpec((_COUT, _COUT), lambda b, i: (0, 0)),
          pl.BlockSpec((1, _COUT), lambda b, i: (0, 0)),
          pl.BlockSpec((1, _COUT), lambda b, i: (0, 0)),
          pl.BlockSpec((1, _COUT), lambda b, i: (0, 0)),
      ],
      out_specs=pl.BlockSpec((1, r, _D), lambda b, i: (b, i, 0)),
      out_shape=jax.ShapeDtypeStruct((_B, n1, _D), jnp.float32),
  )(pos1, ch1, sums1, w2p, w2f, b2, g1, be1)


def _sc_gather(table, idx3):
  ch = idx3.shape[1]
  d = table.shape[1]
  mesh = plsc.VectorSubcoreMesh(core_axis_name="c", subcore_axis_name="s")

  @functools.partial(
      pl.kernel,
      mesh=mesh,
      out_type=jax.ShapeDtypeStruct((_NW * ch * 128, d), jnp.float32),
      scratch_types=[
          pltpu.VMEM((ch, 128), jnp.int32),
          pltpu.VMEM((128, d), jnp.float32),
          pltpu.VMEM((128, d), jnp.float32),
          pltpu.SemaphoreType.DMA,
          pltpu.SemaphoreType.DMA,
      ],
  )
  def k(table_hbm, idx_hbm, out_hbm, idx_v, buf0, buf1, sem0, sem1):
    wid = lax.axis_index("s") * _NC + lax.axis_index("c")
    base = wid * (ch * 128)
    pltpu.sync_copy(idx_hbm.at[wid], idx_v)
    bufs = (buf0, buf1)
    sems = (sem0, sem1)
    handles = [None, None]
    for j in range(ch):
      s = j % 2
      handles[s] = pltpu.async_copy(table_hbm.at[idx_v.at[j]], bufs[s], sems[s])
      if j > 0:
        ps = (j - 1) % 2
        handles[ps].wait()
        pltpu.sync_copy(bufs[ps], out_hbm.at[pl.ds(base + (j - 1) * 128, 128)])
    last = (ch - 1) % 2
    handles[last].wait()
    pltpu.sync_copy(bufs[last], out_hbm.at[pl.ds(base + (ch - 1) * 128, 128)])

  return k(table, idx3)


def _gather_rows(a_flat, idx):
  b, nc, k = idx.shape
  total = b * k * nc
  idx3 = idx.reshape(_NW, total // (_NW * 128), 128)
  out = _sc_gather(a_flat, idx3)
  return out.reshape(b, nc, k, _D)


def _mlp1_body(g_ref, pos_ref, w1p_ref, w_ref, b_ref, out_ref, sums_ref):
  p = pos_ref[0]
  c = jnp.zeros((p.shape[0], _COUT), jnp.float32)
  for d in range(3):
    c = c + p[:, d:d + 1] * w1p_ref[d, :][None, :]
  acc = None
  for kk in range(_K):
    h = jnp.maximum(g_ref[0, :, kk, :_COUT] - c, 0.0)
    y = jnp.dot(h, w_ref[...], preferred_element_type=jnp.float32) + b_ref[...]
    y = jnp.maximum(y, 0.0)
    acc = y if acc is None else jnp.maximum(acc, y)
  out_ref[0] = acc
  first = (pl.program_id(0) == 0) & (pl.program_id(1) == 0)

  @pl.when(first)
  def _():
    sums_ref[...] = jnp.zeros_like(sums_ref)

  ssum = jnp.sum(acc, axis=0, keepdims=True)
  ssq = jnp.sum(acc * acc, axis=0, keepdims=True)
  sums_ref[...] += jnp.concatenate([ssum, ssq], axis=0)


def _mlp1(g, pos, w1p, w11, b11, r=256):
  n1 = _N // 2
  grid = (_B, n1 // r)
  return pl.pallas_call(
      _mlp1_body,
      grid=grid,
      in_specs=[
          pl.BlockSpec((1, r, _K, _D), lambda b, i: (b, i, 0, 0)),
          pl.BlockSpec((1, r, 3), lambda b, i: (b, i, 0)),
          pl.BlockSpec((3, _COUT), lambda b, i: (0, 0)),
          pl.BlockSpec((_COUT, _COUT), lambda b, i: (0, 0)),
          pl.BlockSpec((1, _COUT), lambda b, i: (0, 0)),
      ],
      out_specs=[
          pl.BlockSpec((1, r, _COUT), lambda b, i: (b, i, 0)),
          pl.BlockSpec((2, _COUT), lambda b, i: (0, 0)),
      ],
      out_shape=[
          jax.ShapeDtypeStruct((_B, n1, _COUT), jnp.float32),
          jax.ShapeDtypeStruct((2, _COUT), jnp.float32),
      ],
  )(g, pos, w1p, w11, b11)


def _mlp2_body(g_ref, pos_ref, feat_ref, w2p_ref, w_ref, b_ref, wres_ref,
               ch_ref, res_ref, sums_ref):
  p = pos_ref[0]
  c = jnp.zeros((p.shape[0], _COUT), jnp.float32)
  for d in range(3):
    c = c + p[:, d:d + 1] * w2p_ref[d, :][None, :]
  acc = None
  for kk in range(_K):
    h = jnp.maximum(g_ref[0, :, kk, :_COUT] - c, 0.0)
    y = jnp.dot(h, w_ref[...], preferred_element_type=jnp.float32) + b_ref[...]
    y = jnp.maximum(y, 0.0)
    acc = y if acc is None else jnp.maximum(acc, y)
  ch_ref[0] = acc
  res = jnp.dot(feat_ref[0], wres_ref[...], preferred_element_type=jnp.float32)
  res_ref[0] = res
  first = (pl.program_id(0) == 0) & (pl.program_id(1) == 0)

  @pl.when(first)
  def _():
    sums_ref[...] = jnp.zeros_like(sums_ref)

  sums_ref[...] += jnp.concatenate([
      jnp.sum(acc, axis=0, keepdims=True),
      jnp.sum(acc * acc, axis=0, keepdims=True),
      jnp.sum(res, axis=0, keepdims=True),
      jnp.sum(res * res, axis=0, keepdims=True),
  ], axis=0)


def _mlp2(g, pos, feat, w2p, w21, b21, wres, r=256):
  n2 = _N // 4
  grid = (_B, n2 // r)
  return pl.pallas_call(
      _mlp2_body,
      grid=grid,
      in_specs=[
          pl.BlockSpec((1, r, _K, _D), lambda b, i: (b, i, 0, 0)),
          pl.BlockSpec((1, r, 3), lambda b, i: (b, i, 0)),
          pl.BlockSpec((1, r, _CIN), lambda b, i: (b, i, 0)),
          pl.BlockSpec((3, _COUT), lambda b, i: (0, 0)),
          pl.BlockSpec((_COUT, _COUT), lambda b, i: (0, 0)),
          pl.BlockSpec((1, _COUT), lambda b, i: (0, 0)),
          pl.BlockSpec((_CIN, _COUT), lambda b, i: (0, 0)),
      ],
      out_specs=[
          pl.BlockSpec((1, r, _COUT), lambda b, i: (b, i, 0)),
          pl.BlockSpec((1, r, _COUT), lambda b, i: (b, i, 0)),
          pl.BlockSpec((4, _COUT), lambda b, i: (0, 0)),
      ],
      out_shape=[
          jax.ShapeDtypeStruct((_B, n2, _COUT), jnp.float32),
          jax.ShapeDtypeStruct((_B, n2, _COUT), jnp.float32),
          jax.ShapeDtypeStruct((4, _COUT), jnp.float32),
      ],
  )(g, pos, feat, w2p, w21, b21, wres)


def _fin_body(ch_ref, res_ref, sums_ref, g2_ref, be2_ref, gr_ref, br_ref,
              out_ref, *, count):
  m2 = sums_ref[0, :] / count
  v2 = sums_ref[1, :] / count - m2 * m2
  a2 = g2_ref[0, :] * lax.rsqrt(v2 + _EPS)
  c2 = be2_ref[0, :] - m2 * a2
  mr = sums_ref[2, :] / count
  vr = sums_ref[3, :] / count - mr * mr
  ar = gr_ref[0, :] * lax.rsqrt(vr + _EPS)
  cr = br_ref[0, :] - mr * ar
  out_ref[0] = (ch_ref[0] * a2[None, :] + c2[None, :]
                + res_ref[0] * ar[None, :] + cr[None, :])


def _finalize(ch2, res2, sums, g2, be2, gr, br, r=512):
  n2 = _N // 4
  grid = (_B, n2 // r)
  return pl.pallas_call(
      functools.partial(_fin_body, count=float(_B * n2)),
      grid=grid,
      in_specs=[
          pl.BlockSpec((1, r, _COUT), lambda b, i: (b, i, 0)),
          pl.BlockSpec((1, r, _COUT), lambda b, i: (b, i, 0)),
          pl.BlockSpec((4, _COUT), lambda b, i: (0, 0)),
          pl.BlockSpec((1, _COUT), lambda b, i: (0, 0)),
          pl.BlockSpec((1, _COUT), lambda b, i: (0, 0)),
          pl.BlockSpec((1, _COUT), lambda b, i: (0, 0)),
          pl.BlockSpec((1, _COUT), lambda b, i: (0, 0)),
      ],
      out_specs=pl.BlockSpec((1, r, _COUT), lambda b, i: (b, i, 0)),
      out_shape=jax.ShapeDtypeStruct((_B, n2, _COUT), jnp.float32),
  )(ch2, res2, sums, g2, be2, gr, br)


def kernel(position_matrix, channel_matrix, W1_0, b1_0, W1_1, b1_1, W_res,
           W2_0, b2_0, W2_1, b2_1, gamma1, beta1, gamma2, beta2, gamma_r,
           beta_r, n_select_0, n_select_1, n_select_2):
  n0, n1, n2 = _N, _N // 2, _N // 4
  zero = ((jnp.asarray(n_select_0) - n0)
          + (jnp.asarray(n_select_1) - n1)
          + (jnp.asarray(n_select_2) - n2)).astype(position_matrix.dtype)

  pos = position_matrix
  feat = channel_matrix
  w1p, w1f = W1_0[:3], W1_0[3:]
  w2p, w2f = W2_0[:3], W2_0[3:]
  row = lambda v: v.reshape(1, _COUT)

  idx1 = _topk(pos, n0, n1)
  a1 = _build_a1(pos, feat, w1p, w1f, row(b1_0))
  g1 = _gather_rows(a1.reshape(_B * n0, _D), idx1)
  ch1, sums1 = _mlp1(g1, pos, w1p, W1_1, row(b1_1))

  pos1 = pos[:, :n1]
  idx2 = _topk(pos1, n1, n2)
  a2 = _build_a2(pos1, ch1, sums1, w2p, w2f, row(b2_0), row(gamma1),
                 row(beta1))
  g2 = _gather_rows(a2.reshape(_B * n1, _D), idx2)
  ch2_raw, res2, sums2 = _mlp2(g2, pos1, feat[:, :n2], w2p, W2_1, row(b2_1),
                               W_res)
  ch2 = _finalize(ch2_raw, res2, sums2, row(gamma2), row(beta2), row(gamma_r),
                  row(beta_r))
  pos2 = pos[:, :n2] + zero
  return (pos2, ch2)


# speedup vs baseline: 23.6073x; 1.0160x over previous
import functools

import jax
import jax.numpy as jnp
from jax import lax
from jax.experimental import pallas as pl
from jax.experimental.pallas import tpu as pltpu
from jax.experimental.pallas import tpu_sc as plsc

_B = 2
_N = 8192
_K = 16
_CIN = 32
_COUT = 64
_EPS = 1e-5

_NC = 2
_NS = 16
_NW = _NC * _NS
_D = 128


def _score_body(pos_ref, posT_ref, s_ref, seg_ref, *, npts, rs):
  d2 = None
  for d in range(3):
    cd = pos_ref[0][:, d:d + 1]
    pd = posT_ref[0, d, :][None, :]
    diff = cd - pd
    sq = diff * diff
    d2 = sq if d2 is None else d2 + sq
  s = -d2
  s_ref[0] = s
  nseg = npts // 128
  for g in range(nseg):
    seg_ref[0, :, g] = jnp.max(s[:, g * 128:(g + 1) * 128], axis=1)


def _scores(pos, npts, nctr, rs=128):
  posT = jnp.swapaxes(pos, 1, 2)
  nseg = npts // 128
  grid = (_B, nctr // rs)
  return pl.pallas_call(
      functools.partial(_score_body, npts=npts, rs=rs),
      grid=grid,
      in_specs=[
          pl.BlockSpec((1, rs, 3), lambda b, i: (b, i, 0)),
          pl.BlockSpec((1, 3, npts), lambda b, i: (b, 0, 0)),
      ],
      out_specs=[
          pl.BlockSpec((1, rs, npts), lambda b, i: (b, i, 0)),
          pl.BlockSpec((1, rs, nseg), lambda b, i: (b, i, 0)),
      ],
      out_shape=[
          jax.ShapeDtypeStruct((_B, nctr, npts), jnp.float32),
          jax.ShapeDtypeStruct((_B, nctr, nseg), jnp.float32),
      ],
  )(pos, posT)


def _sc_extract(scores2, seg2, npts, nctr):
  rt = _B * nctr
  nseg = npts // 128
  q = nseg // 16
  rows_per = rt // _NW
  nit = rows_per // 2
  sh = nctr.bit_length() - 1
  neg = jnp.float32(-3.0e38)
  bigi = jnp.int32(1 << 20)
  mesh = plsc.VectorSubcoreMesh(core_axis_name="c", subcore_axis_name="s")

  @functools.partial(
      pl.kernel,
      mesh=mesh,
      out_type=jax.ShapeDtypeStruct((rt * _K,), jnp.int32),
      scratch_types=[
          pltpu.VMEM((npts,), jnp.float32),
          pltpu.VMEM((npts,), jnp.float32),
          pltpu.VMEM((rows_per, nseg), jnp.float32),
          pltpu.VMEM((2 * _K,), jnp.int32),
          pltpu.SemaphoreType.DMA,
          pltpu.SemaphoreType.DMA,
      ],
  )
  def k(s_hbm, g_hbm, o_hbm, row0, row1, segall, idxst, sem0, sem1):
    wid = lax.axis_index("s") * _NC + lax.axis_index("c")
    base = wid * rows_per
    pltpu.sync_copy(g_hbm.at[pl.ds(base, rows_per)], segall)
    iota16 = lax.iota(jnp.int32, 16)
    dnums = lax.GatherDimensionNumbers(
        offset_dims=(), collapsed_slice_dims=(0,), start_index_map=(0,))

    def vperm(x, perm):
      return lax.gather(x, perm[:, None], dnums, slice_sizes=(1,),
                        mode=lax.GatherScatterMode.PROMISE_IN_BOUNDS)

    def bfly(x, op):
      for s in (8, 4, 2, 1):
        x = op(x, vperm(x, iota16 ^ s))
      return x

    def to_scalar(splat_i32):
      return splat_i32[0]

    def extract_row(row_ref, rloc, rglob):
      gbase = (rglob >> sh) * npts
      sv = [segall[rloc, pl.ds(qq * 16, 16)] for qq in range(q)]
      idxacc = jnp.zeros((16,), jnp.int32)
      for t in range(_K):
        m = sv[0]
        for qq in range(1, q):
          m = jnp.maximum(m, sv[qq])
        m = bfly(m, jnp.maximum)
        pk = None
        for qq in range(q):
          cand = jnp.where(sv[qq] == m, iota16 + qq * 16, bigi)
          pk = cand if pk is None else jnp.minimum(pk, cand)
        sstar = bfly(pk, jnp.minimum)
        col0 = to_scalar(sstar) * 128
        ch = [row_ref[pl.ds(col0 + v * 16, 16)] for v in range(8)]
        jp = None
        for v in range(8):
          cand = jnp.where(ch[v] == m, iota16 + v * 16, bigi)
          jp = cand if jp is None else jnp.minimum(jp, cand)
        jloc = bfly(jp, jnp.minimum)
        jloc_s = to_scalar(jloc)
        cb = col0 + (jloc_s >> 4) * 16
        cv = row_ref[pl.ds(cb, 16)]
        row_ref[pl.ds(cb, 16)] = jnp.where(iota16 == (jloc_s & 15), neg, cv)
        nm = None
        for v in range(8):
          cm = jnp.where(iota16 + v * 16 == jloc, neg, ch[v])
          nm = cm if nm is None else jnp.maximum(nm, cm)
        nms = bfly(nm, jnp.maximum)
        for qq in range(q):
          sv[qq] = jnp.where(iota16 + qq * 16 == sstar, nms, sv[qq])
        idxacc = jnp.where(iota16 == t,
                           gbase + col0 + (jloc & jnp.int32(127)), idxacc)
      return idxacc

    pltpu.sync_copy(s_hbm.at[base], row0)
    pltpu.async_copy(s_hbm.at[base + 1], row1, sem1)

    def body(i, _):
      r0 = base + 2 * i
      idxst[pl.ds(0, 16)] = extract_row(row0, 2 * i, r0)
      nxt0 = jnp.minimum(r0 + 2, rt - 1)
      pltpu.async_copy(s_hbm.at[nxt0], row0, sem0)
      pltpu.make_async_copy(s_hbm.at[base], row1, sem1).wait()
      idxst[pl.ds(16, 16)] = extract_row(row1, 2 * i + 1, r0 + 1)
      nxt1 = jnp.minimum(r0 + 3, rt - 1)
      pltpu.async_copy(s_hbm.at[nxt1], row1, sem1)
      pltpu.sync_copy(idxst, o_hbm.at[pl.ds(r0 * _K, 2 * _K)])
      pltpu.make_async_copy(s_hbm.at[base], row0, sem0).wait()
      return 0

    lax.fori_loop(0, nit, body, 0)
    pltpu.make_async_copy(s_hbm.at[base], row1, sem1).wait()

  return k(scores2, seg2)


def _topk(pos, npts, nctr):
  s, seg = _scores(pos, npts, nctr)
  idx = _sc_extract(s.reshape(_B * nctr, npts),
                    seg.reshape(_B * nctr, npts // 128), npts, nctr)
  return idx.reshape(_B, nctr, _K)


def _a1_body(pos_ref, feat_ref, w1p_ref, w1f_ref, b1_ref, out_ref):
  p = pos_ref[0]
  f = feat_ref[0]
  acc = jnp.dot(f, w1f_ref[...], preferred_element_type=jnp.float32)
  for d in range(3):
    acc = acc + p[:, d:d + 1] * w1p_ref[d, :][None, :]
  acc = acc + b1_ref[...]
  out_ref[0] = jnp.concatenate(
      [acc, jnp.zeros((acc.shape[0], _D - _COUT), jnp.float32)], axis=1)


def _build_a1(pos, feat, w1p, w1f, b1, r=512):
  grid = (_B, _N // r)
  return pl.pallas_call(
      _a1_body,
      grid=grid,
      in_specs=[
          pl.BlockSpec((1, r, 3), lambda b, i: (b, i, 0)),
          pl.BlockSpec((1, r, _CIN), lambda b, i: (b, i, 0)),
          pl.BlockSpec((3, _COUT), lambda b, i: (0, 0)),
          pl.BlockSpec((_CIN, _COUT), lambda b, i: (0, 0)),
          pl.BlockSpec((1, _COUT), lambda b, i: (0, 0)),
      ],
      out_specs=pl.BlockSpec((1, r, _D), lambda b, i: (b, i, 0)),
      out_shape=jax.ShapeDtypeStruct((_B, _N, _D), jnp.float32),
  )(pos, feat, w1p, w1f, b1)


def _a2_body(pos_ref, ch_ref, sums_ref, w2p_ref, w2f_ref, b2_ref, g1_ref,
             be1_ref, out_ref, *, count):
  mean = sums_ref[0, :] / count
  var = sums_ref[1, :] / count - mean * mean
  a1 = g1_ref[0, :] * lax.rsqrt(var + _EPS)
  c1 = be1_ref[0, :] - mean * a1
  ch = ch_ref[0]
  chbn = ch * a1[None, :] + c1[None, :]
  acc = jnp.dot(chbn, w2f_ref[...], preferred_element_type=jnp.float32)
  p = pos_ref[0]
  for d in range(3):
    acc = acc + p[:, d:d + 1] * w2p_ref[d, :][None, :]
  acc = acc + b2_ref[...]
  out_ref[0] = jnp.concatenate(
      [acc, jnp.zeros((acc.shape[0], _D - _COUT), jnp.float32)], axis=1)


def _build_a2(pos1, ch1, sums1, w2p, w2f, b2, g1, be1, r=512):
  n1 = _N // 2
  grid = (_B, n1 // r)
  return pl.pallas_call(
      functools.partial(_a2_body, count=float(_B * n1)),
      grid=grid,
      in_specs=[
          pl.BlockSpec((1, r, 3), lambda b, i: (b, i, 0)),
          pl.BlockSpec((1, r, _COUT), lambda b, i: (b, i, 0)),
          pl.BlockSpec((2, _COUT), lambda b, i: (0, 0)),
          pl.BlockSpec((3, _COUT), lambda b, i: (0, 0)),
          pl.BlockSpec((_COUT, _COUT), lambda b, i: (0, 0)),
          pl.BlockSpec((1, _COUT), lambda b, i: (0, 0)),
          pl.BlockSpec((1, _COUT), lambda b, i: (0, 0)),
          pl.BlockSpec((1, _COUT), lambda b, i: (0, 0)),
      ],
      out_specs=pl.BlockSpec((1, r, _D), lambda b, i: (b, i, 0)),
      out_shape=jax.ShapeDtypeStruct((_B, n1, _D), jnp.float32),
  )(pos1, ch1, sums1, w2p, w2f, b2, g1, be1)


def _sc_gather(table, idx3):
  ch = idx3.shape[1]
  d = table.shape[1]
  mesh = plsc.VectorSubcoreMesh(core_axis_name="c", subcore_axis_name="s")

  @functools.partial(
      pl.kernel,
      mesh=mesh,
      out_type=jax.ShapeDtypeStruct((_NW * ch * 128, d), jnp.float32),
      scratch_types=[
          pltpu.VMEM((ch, 128), jnp.int32),
          pltpu.VMEM((128, d), jnp.float32),
          pltpu.VMEM((128, d), jnp.float32),
          pltpu.SemaphoreType.DMA,
          pltpu.SemaphoreType.DMA,
      ],
  )
  def k(table_hbm, idx_hbm, out_hbm, idx_v, buf0, buf1, sem0, sem1):
    wid = lax.axis_index("s") * _NC + lax.axis_index("c")
    base = wid * (ch * 128)
    pltpu.sync_copy(idx_hbm.at[wid], idx_v)
    bufs = (buf0, buf1)
    sems = (sem0, sem1)
    handles = [None, None]
    for j in range(ch):
      s = j % 2
      handles[s] = pltpu.async_copy(table_hbm.at[idx_v.at[j]], bufs[s], sems[s])
      if j > 0:
        ps = (j - 1) % 2
        handles[ps].wait()
        pltpu.sync_copy(bufs[ps], out_hbm.at[pl.ds(base + (j - 1) * 128, 128)])
    last = (ch - 1) % 2
    handles[last].wait()
    pltpu.sync_copy(bufs[last], out_hbm.at[pl.ds(base + (ch - 1) * 128, 128)])

  return k(table, idx3)


def _gather_rows(a_flat, idx):
  b, nc, k = idx.shape
  total = b * k * nc
  idx3 = idx.reshape(_NW, total // (_NW * 128), 128)
  out = _sc_gather(a_flat, idx3)
  return out.reshape(b, nc, k, _D)


def _mlp1_body(g_ref, pos_ref, w1p_ref, w_ref, b_ref, out_ref, sums_ref):
  p = pos_ref[0]
  r = p.shape[0]
  c = jnp.zeros((r, _COUT), jnp.float32)
  for d in range(3):
    c = c + p[:, d:d + 1] * w1p_ref[d, :][None, :]
  g = g_ref[0, :, :, :_COUT]
  h = jnp.maximum(g - c[:, None, :], 0.0).reshape(r * _K, _COUT)
  y = jnp.dot(h, w_ref[...], preferred_element_type=jnp.float32) + b_ref[...]
  y = jnp.maximum(y, 0.0)
  acc = jnp.max(y.reshape(r, _K, _COUT), axis=1)
  out_ref[0] = acc
  first = (pl.program_id(0) == 0) & (pl.program_id(1) == 0)

  @pl.when(first)
  def _():
    sums_ref[...] = jnp.zeros_like(sums_ref)

  ssum = jnp.sum(acc, axis=0, keepdims=True)
  ssq = jnp.sum(acc * acc, axis=0, keepdims=True)
  sums_ref[...] += jnp.concatenate([ssum, ssq], axis=0)


def _mlp1(g, pos, w1p, w11, b11, r=256):
  n1 = _N // 2
  grid = (_B, n1 // r)
  return pl.pallas_call(
      _mlp1_body,
      grid=grid,
      in_specs=[
          pl.BlockSpec((1, r, _K, _D), lambda b, i: (b, i, 0, 0)),
          pl.BlockSpec((1, r, 3), lambda b, i: (b, i, 0)),
          pl.BlockSpec((3, _COUT), lambda b, i: (0, 0)),
          pl.BlockSpec((_COUT, _COUT), lambda b, i: (0, 0)),
          pl.BlockSpec((1, _COUT), lambda b, i: (0, 0)),
      ],
      out_specs=[
          pl.BlockSpec((1, r, _COUT), lambda b, i: (b, i, 0)),
          pl.BlockSpec((2, _COUT), lambda b, i: (0, 0)),
      ],
      out_shape=[
          jax.ShapeDtypeStruct((_B, n1, _COUT), jnp.float32),
          jax.ShapeDtypeStruct((2, _COUT), jnp.float32),
      ],
  )(g, pos, w1p, w11, b11)


def _mlp2_body(g_ref, pos_ref, feat_ref, w2p_ref, w_ref, b_ref, wres_ref,
               ch_ref, res_ref, sums_ref):
  p = pos_ref[0]
  r = p.shape[0]
  c = jnp.zeros((r, _COUT), jnp.float32)
  for d in range(3):
    c = c + p[:, d:d + 1] * w2p_ref[d, :][None, :]
  g = g_ref[0, :, :, :_COUT]
  h = jnp.maximum(g - c[:, None, :], 0.0).reshape(r * _K, _COUT)
  y = jnp.dot(h, w_ref[...], preferred_element_type=jnp.float32) + b_ref[...]
  y = jnp.maximum(y, 0.0)
  acc = jnp.max(y.reshape(r, _K, _COUT), axis=1)
  ch_ref[0] = acc
  res = jnp.dot(feat_ref[0], wres_ref[...], preferred_element_type=jnp.float32)
  res_ref[0] = res
  first = (pl.program_id(0) == 0) & (pl.program_id(1) == 0)

  @pl.when(first)
  def _():
    sums_ref[...] = jnp.zeros_like(sums_ref)

  sums_ref[...] += jnp.concatenate([
      jnp.sum(acc, axis=0, keepdims=True),
      jnp.sum(acc * acc, axis=0, keepdims=True),
      jnp.sum(res, axis=0, keepdims=True),
      jnp.sum(res * res, axis=0, keepdims=True),
  ], axis=0)


def _mlp2(g, pos, feat, w2p, w21, b21, wres, r=256):
  n2 = _N // 4
  grid = (_B, n2 // r)
  return pl.pallas_call(
      _mlp2_body,
      grid=grid,
      in_specs=[
          pl.BlockSpec((1, r, _K, _D), lambda b, i: (b, i, 0, 0)),
          pl.BlockSpec((1, r, 3), lambda b, i: (b, i, 0)),
          pl.BlockSpec((1, r, _CIN), lambda b, i: (b, i, 0)),
          pl.BlockSpec((3, _COUT), lambda b, i: (0, 0)),
          pl.BlockSpec((_COUT, _COUT), lambda b, i: (0, 0)),
          pl.BlockSpec((1, _COUT), lambda b, i: (0, 0)),
          pl.BlockSpec((_CIN, _COUT), lambda b, i: (0, 0)),
      ],
      out_specs=[
          pl.BlockSpec((1, r, _COUT), lambda b, i: (b, i, 0)),
          pl.BlockSpec((1, r, _COUT), lambda b, i: (b, i, 0)),
          pl.BlockSpec((4, _COUT), lambda b, i: (0, 0)),
      ],
      out_shape=[
          jax.ShapeDtypeStruct((_B, n2, _COUT), jnp.float32),
          jax.ShapeDtypeStruct((_B, n2, _COUT), jnp.float32),
          jax.ShapeDtypeStruct((4, _COUT), jnp.float32),
      ],
  )(g, pos, feat, w2p, w21, b21, wres)


def _fin_body(ch_ref, res_ref, sums_ref, g2_ref, be2_ref, gr_ref, br_ref,
              out_ref, *, count):
  m2 = sums_ref[0, :] / count
  v2 = sums_ref[1, :] / count - m2 * m2
  a2 = g2_ref[0, :] * lax.rsqrt(v2 + _EPS)
  c2 = be2_ref[0, :] - m2 * a2
  mr = sums_ref[2, :] / count
  vr = sums_ref[3, :] / count - mr * mr
  ar = gr_ref[0, :] * lax.rsqrt(vr + _EPS)
  cr = br_ref[0, :] - mr * ar
  out_ref[0] = (ch_ref[0] * a2[None, :] + c2[None, :]
                + res_ref[0] * ar[None, :] + cr[None, :])


def _finalize(ch2, res2, sums, g2, be2, gr, br, r=512):
  n2 = _N // 4
  grid = (_B, n2 // r)
  return pl.pallas_call(
      functools.partial(_fin_body, count=float(_B * n2)),
      grid=grid,
      in_specs=[
          pl.BlockSpec((1, r, _COUT), lambda b, i: (b, i, 0)),
          pl.BlockSpec((1, r, _COUT), lambda b, i: (b, i, 0)),
          pl.BlockSpec((4, _COUT), lambda b, i: (0, 0)),
          pl.BlockSpec((1, _COUT), lambda b, i: (0, 0)),
          pl.BlockSpec((1, _COUT), lambda b, i: (0, 0)),
          pl.BlockSpec((1, _COUT), lambda b, i: (0, 0)),
          pl.BlockSpec((1, _COUT), lambda b, i: (0, 0)),
      ],
      out_specs=pl.BlockSpec((1, r, _COUT), lambda b, i: (b, i, 0)),
      out_shape=jax.ShapeDtypeStruct((_B, n2, _COUT), jnp.float32),
  )(ch2, res2, sums, g2, be2, gr, br)


def kernel(position_matrix, channel_matrix, W1_0, b1_0, W1_1, b1_1, W_res,
           W2_0, b2_0, W2_1, b2_1, gamma1, beta1, gamma2, beta2, gamma_r,
           beta_r, n_select_0, n_select_1, n_select_2):
  n0, n1, n2 = _N, _N // 2, _N // 4
  zero = ((jnp.asarray(n_select_0) - n0)
          + (jnp.asarray(n_select_1) - n1)
          + (jnp.asarray(n_select_2) - n2)).astype(position_matrix.dtype)

  pos = position_matrix
  feat = channel_matrix
  w1p, w1f = W1_0[:3], W1_0[3:]
  w2p, w2f = W2_0[:3], W2_0[3:]
  row = lambda v: v.reshape(1, _COUT)

  idx1 = _topk(pos, n0, n1)
  a1 = _build_a1(pos, feat, w1p, w1f, row(b1_0))
  g1 = _gather_rows(a1.reshape(_B * n0, _D), idx1)
  ch1, sums1 = _mlp1(g1, pos, w1p, W1_1, row(b1_1))

  pos1 = pos[:, :n1]
  idx2 = _topk(pos1, n1, n2)
  a2 = _build_a2(pos1, ch1, sums1, w2p, w2f, row(b2_0), row(gamma1),
                 row(beta1))
  g2 = _gather_rows(a2.reshape(_B * n1, _D), idx2)
  ch2_raw, res2, sums2 = _mlp2(g2, pos1, feat[:, :n2], w2p, W2_1, row(b2_1),
                               W_res)
  ch2 = _finalize(ch2_raw, res2, sums2, row(gamma2), row(beta2), row(gamma_r),
                  row(beta_r))
  pos2 = pos[:, :n2] + zero
  return (pos2, ch2)


# speedup vs baseline: 24.6356x; 1.0436x over previous
import functools

import jax
import jax.numpy as jnp
from jax import lax
from jax.experimental import pallas as pl
from jax.experimental.pallas import tpu as pltpu
from jax.experimental.pallas import tpu_sc as plsc

_B = 2
_N = 8192
_K = 16
_CIN = 32
_COUT = 64
_EPS = 1e-5

_NC = 2
_NS = 16
_NW = _NC * _NS
_D = 128


def _score_body(pos_ref, posT_ref, s_ref, seg_ref, *, npts, rs):
  d2 = None
  for d in range(3):
    cd = pos_ref[0][:, d:d + 1]
    pd = posT_ref[0, d, :][None, :]
    diff = cd - pd
    sq = diff * diff
    d2 = sq if d2 is None else d2 + sq
  s = -d2
  s_ref[0] = s
  nseg = npts // 128
  for g in range(nseg):
    seg_ref[0, :, g] = jnp.max(s[:, g * 128:(g + 1) * 128], axis=1)


def _scores(pos, npts, nctr, rs=128):
  posT = jnp.swapaxes(pos, 1, 2)
  nseg = npts // 128
  grid = (_B, nctr // rs)
  return pl.pallas_call(
      functools.partial(_score_body, npts=npts, rs=rs),
      grid=grid,
      in_specs=[
          pl.BlockSpec((1, rs, 3), lambda b, i: (b, i, 0)),
          pl.BlockSpec((1, 3, npts), lambda b, i: (b, 0, 0)),
      ],
      out_specs=[
          pl.BlockSpec((1, rs, npts), lambda b, i: (b, i, 0)),
          pl.BlockSpec((1, rs, nseg), lambda b, i: (b, i, 0)),
      ],
      out_shape=[
          jax.ShapeDtypeStruct((_B, nctr, npts), jnp.float32),
          jax.ShapeDtypeStruct((_B, nctr, nseg), jnp.float32),
      ],
  )(pos, posT)


def _sc_extract(scores2, seg2, npts, nctr):
  rt = _B * nctr
  nseg = npts // 128
  q = nseg // 16
  rows_per = rt // _NW
  nit = rows_per // 2
  sh = nctr.bit_length() - 1
  neg = jnp.float32(-3.0e38)
  bigi = jnp.int32(1 << 20)
  mesh = plsc.VectorSubcoreMesh(core_axis_name="c", subcore_axis_name="s")

  @functools.partial(
      pl.kernel,
      mesh=mesh,
      out_type=jax.ShapeDtypeStruct((rt * _K,), jnp.int32),
      scratch_types=[
          pltpu.VMEM((npts,), jnp.float32),
          pltpu.VMEM((npts,), jnp.float32),
          pltpu.VMEM((npts,), jnp.float32),
          pltpu.VMEM((npts,), jnp.float32),
          pltpu.VMEM((rows_per, nseg), jnp.float32),
          pltpu.VMEM((2 * _K,), jnp.int32),
          pltpu.SemaphoreType.DMA,
          pltpu.SemaphoreType.DMA,
          pltpu.SemaphoreType.DMA,
          pltpu.SemaphoreType.DMA,
      ],
  )
  def k(s_hbm, g_hbm, o_hbm, ra0, rb0, ra1, rb1, segall, idxst,
        sa0, sb0, sa1, sb1):
    wid = lax.axis_index("s") * _NC + lax.axis_index("c")
    base = wid * rows_per
    pltpu.sync_copy(g_hbm.at[pl.ds(base, rows_per)], segall)
    iota16 = lax.iota(jnp.int32, 16)
    dnums = lax.GatherDimensionNumbers(
        offset_dims=(), collapsed_slice_dims=(0,), start_index_map=(0,))

    def vperm(x, perm):
      return lax.gather(x, perm[:, None], dnums, slice_sizes=(1,),
                        mode=lax.GatherScatterMode.PROMISE_IN_BOUNDS)

    def bfly(x, op):
      for s in (8, 4, 2, 1):
        x = op(x, vperm(x, iota16 ^ s))
      return x

    def extract_pair(refs, rlocs, rglobs):
      gb = [(rg >> sh) * npts for rg in rglobs]
      sv = [[segall[rl, pl.ds(qq * 16, 16)] for qq in range(q)]
            for rl in rlocs]
      acc = [jnp.zeros((16,), jnp.int32) for _ in refs]
      for t in range(_K):
        m = []
        for j in range(2):
          mj = sv[j][0]
          for qq in range(1, q):
            mj = jnp.maximum(mj, sv[j][qq])
          m.append(mj)
        m = [bfly(x, jnp.maximum) for x in m]
        pk = []
        for j in range(2):
          pkj = None
          for qq in range(q):
            cand = jnp.where(sv[j][qq] == m[j], iota16 + qq * 16, bigi)
            pkj = cand if pkj is None else jnp.minimum(pkj, cand)
          pk.append(pkj)
        ss = [bfly(x, jnp.minimum) for x in pk]
        c0 = [x[0] * 128 for x in ss]
        ch = [[refs[j][pl.ds(c0[j] + v * 16, 16)] for v in range(8)]
              for j in range(2)]
        jp = []
        for j in range(2):
          jpj = None
          for v in range(8):
            cand = jnp.where(ch[j][v] == m[j], iota16 + v * 16, bigi)
            jpj = cand if jpj is None else jnp.minimum(jpj, cand)
          jp.append(jpj)
        jl = [bfly(x, jnp.minimum) for x in jp]
        js = [x[0] for x in jl]
        for j in range(2):
          cb = c0[j] + (js[j] >> 4) * 16
          cv = refs[j][pl.ds(cb, 16)]
          refs[j][pl.ds(cb, 16)] = jnp.where(iota16 == (js[j] & 15), neg, cv)
        nm = []
        for j in range(2):
          nmj = None
          for v in range(8):
            cm = jnp.where(iota16 + v * 16 == jl[j], neg, ch[j][v])
            nmj = cm if nmj is None else jnp.maximum(nmj, cm)
          nm.append(nmj)
        ns = [bfly(x, jnp.maximum) for x in nm]
        for j in range(2):
          for qq in range(q):
            sv[j][qq] = jnp.where(iota16 + qq * 16 == ss[j], ns[j], sv[j][qq])
          acc[j] = jnp.where(iota16 == t,
                             gb[j] + c0[j] + (jl[j] & jnp.int32(127)), acc[j])
      return acc

    pltpu.async_copy(s_hbm.at[base + 0], ra0, sa0)
    pltpu.async_copy(s_hbm.at[base + 1], rb0, sb0)
    pltpu.async_copy(s_hbm.at[base + 2], ra1, sa1)
    pltpu.async_copy(s_hbm.at[base + 3], rb1, sb1)

    def do_pair(r0, rl0, ra, rb, sa, sb):
      pltpu.make_async_copy(s_hbm.at[base], ra, sa).wait()
      pltpu.make_async_copy(s_hbm.at[base], rb, sb).wait()
      aa, ab = extract_pair((ra, rb), (rl0, rl0 + 1), (r0, r0 + 1))
      idxst[pl.ds(0, 16)] = aa
      idxst[pl.ds(16, 16)] = ab
      pltpu.async_copy(s_hbm.at[jnp.minimum(r0 + 4, rt - 1)], ra, sa)
      pltpu.async_copy(s_hbm.at[jnp.minimum(r0 + 5, rt - 1)], rb, sb)
      pltpu.sync_copy(idxst, o_hbm.at[pl.ds(r0 * _K, 2 * _K)])

    def body(i, _):
      r0 = base + 4 * i
      do_pair(r0, 4 * i, ra0, rb0, sa0, sb0)
      do_pair(r0 + 2, 4 * i + 2, ra1, rb1, sa1, sb1)
      return 0

    lax.fori_loop(0, rows_per // 4, body, 0)
    pltpu.make_async_copy(s_hbm.at[base], ra0, sa0).wait()
    pltpu.make_async_copy(s_hbm.at[base], rb0, sb0).wait()
    pltpu.make_async_copy(s_hbm.at[base], ra1, sa1).wait()
    pltpu.make_async_copy(s_hbm.at[base], rb1, sb1).wait()

  return k(scores2, seg2)


def _topk(pos, npts, nctr):
  s, seg = _scores(pos, npts, nctr)
  idx = _sc_extract(s.reshape(_B * nctr, npts),
                    seg.reshape(_B * nctr, npts // 128), npts, nctr)
  return idx.reshape(_B, nctr, _K)


def _a1_body(pos_ref, feat_ref, w1p_ref, w1f_ref, b1_ref, out_ref):
  p = pos_ref[0]
  f = feat_ref[0]
  acc = jnp.dot(f, w1f_ref[...], preferred_element_type=jnp.float32)
  for d in range(3):
    acc = acc + p[:, d:d + 1] * w1p_ref[d, :][None, :]
  acc = acc + b1_ref[...]
  out_ref[0] = jnp.concatenate(
      [acc, jnp.zeros((acc.shape[0], _D - _COUT), jnp.float32)], axis=1)


def _build_a1(pos, feat, w1p, w1f, b1, r=512):
  grid = (_B, _N // r)
  return pl.pallas_call(
      _a1_body,
      grid=grid,
      in_specs=[
          pl.BlockSpec((1, r, 3), lambda b, i: (b, i, 0)),
          pl.BlockSpec((1, r, _CIN), lambda b, i: (b, i, 0)),
          pl.BlockSpec((3, _COUT), lambda b, i: (0, 0)),
          pl.BlockSpec((_CIN, _COUT), lambda b, i: (0, 0)),
          pl.BlockSpec((1, _COUT), lambda b, i: (0, 0)),
      ],
      out_specs=pl.BlockSpec((1, r, _D), lambda b, i: (b, i, 0)),
      out_shape=jax.ShapeDtypeStruct((_B, _N, _D), jnp.float32),
  )(pos, feat, w1p, w1f, b1)


def _a2_body(pos_ref, ch_ref, sums_ref, w2p_ref, w2f_ref, b2_ref, g1_ref,
             be1_ref, out_ref, *, count):
  mean = sums_ref[0, :] / count
  var = sums_ref[1, :] / count - mean * mean
  a1 = g1_ref[0, :] * lax.rsqrt(var + _EPS)
  c1 = be1_ref[0, :] - mean * a1
  ch = ch_ref[0]
  chbn = ch * a1[None, :] + c1[None, :]
  acc = jnp.dot(chbn, w2f_ref[...], preferred_element_type=jnp.float32)
  p = pos_ref[0]
  for d in range(3):
    acc = acc + p[:, d:d + 1] * w2p_ref[d, :][None, :]
  acc = acc + b2_ref[...]
  out_ref[0] = jnp.concatenate(
      [acc, jnp.zeros((acc.shape[0], _D - _COUT), jnp.float32)], axis=1)


def _build_a2(pos1, ch1, sums1, w2p, w2f, b2, g1, be1, r=512):
  n1 = _N // 2
  grid = (_B, n1 // r)
  return pl.pallas_call(
      functools.partial(_a2_body, count=float(_B * n1)),
      grid=grid,
      in_specs=[
          pl.BlockSpec((1, r, 3), lambda b, i: (b, i, 0)),
          pl.BlockSpec((1, r, _COUT), lambda b, i: (b, i, 0)),
          pl.BlockSpec((2, _COUT), lambda b, i: (0, 0)),
          pl.BlockSpec((3, _COUT), lambda b, i: (0, 0)),
          pl.BlockSpec((_COUT, _COUT), lambda b, i: (0, 0)),
          pl.BlockSpec((1, _COUT), lambda b, i: (0, 0)),
          pl.BlockSpec((1, _COUT), lambda b, i: (0, 0)),
          pl.BlockSpec((1, _COUT), lambda b, i: (0, 0)),
      ],
      out_specs=pl.BlockSpec((1, r, _D), lambda b, i: (b, i, 0)),
      out_shape=jax.ShapeDtypeStruct((_B, n1, _D), jnp.float32),
  )(pos1, ch1, sums1, w2p, w2f, b2, g1, be1)


def _sc_gather(table, idx3):
  ch = idx3.shape[1]
  d = table.shape[1]
  mesh = plsc.VectorSubcoreMesh(core_axis_name="c", subcore_axis_name="s")

  @functools.partial(
      pl.kernel,
      mesh=mesh,
      out_type=jax.ShapeDtypeStruct((_NW * ch * 128, d), jnp.float32),
      scratch_types=[
          pltpu.VMEM((ch, 128), jnp.int32),
          pltpu.VMEM((128, d), jnp.float32),
          pltpu.VMEM((128, d), jnp.float32),
          pltpu.SemaphoreType.DMA,
          pltpu.SemaphoreType.DMA,
      ],
  )
  def k(table_hbm, idx_hbm, out_hbm, idx_v, buf0, buf1, sem0, sem1):
    wid = lax.axis_index("s") * _NC + lax.axis_index("c")
    base = wid * (ch * 128)
    pltpu.sync_copy(idx_hbm.at[wid], idx_v)
    bufs = (buf0, buf1)
    sems = (sem0, sem1)
    handles = [None, None]
    for j in range(ch):
      s = j % 2
      handles[s] = pltpu.async_copy(table_hbm.at[idx_v.at[j]], bufs[s], sems[s])
      if j > 0:
        ps = (j - 1) % 2
        handles[ps].wait()
        pltpu.sync_copy(bufs[ps], out_hbm.at[pl.ds(base + (j - 1) * 128, 128)])
    last = (ch - 1) % 2
    handles[last].wait()
    pltpu.sync_copy(bufs[last], out_hbm.at[pl.ds(base + (ch - 1) * 128, 128)])

  return k(table, idx3)


def _gather_rows(a_flat, idx):
  b, nc, k = idx.shape
  total = b * k * nc
  idx3 = idx.reshape(_NW, total // (_NW * 128), 128)
  out = _sc_gather(a_flat, idx3)
  return out.reshape(b, nc, k, _D)


def _mlp1_body(g_ref, pos_ref, w1p_ref, w_ref, b_ref, out_ref, sums_ref):
  p = pos_ref[0]
  r = p.shape[0]
  c = jnp.zeros((r, _COUT), jnp.float32)
  for d in range(3):
    c = c + p[:, d:d + 1] * w1p_ref[d, :][None, :]
  g = g_ref[0, :, :, :_COUT]
  h = jnp.maximum(g - c[:, None, :], 0.0).reshape(r * _K, _COUT)
  y = jnp.dot(h, w_ref[...], preferred_element_type=jnp.float32) + b_ref[...]
  y = jnp.maximum(y, 0.0)
  acc = jnp.max(y.reshape(r, _K, _COUT), axis=1)
  out_ref[0] = acc
  first = (pl.program_id(0) == 0) & (pl.program_id(1) == 0)

  @pl.when(first)
  def _():
    sums_ref[...] = jnp.zeros_like(sums_ref)

  ssum = jnp.sum(acc, axis=0, keepdims=True)
  ssq = jnp.sum(acc * acc, axis=0, keepdims=True)
  sums_ref[...] += jnp.concatenate([ssum, ssq], axis=0)


def _mlp1(g, pos, w1p, w11, b11, r=256):
  n1 = _N // 2
  grid = (_B, n1 // r)
  return pl.pallas_call(
      _mlp1_body,
      grid=grid,
      in_specs=[
          pl.BlockSpec((1, r, _K, _D), lambda b, i: (b, i, 0, 0)),
          pl.BlockSpec((1, r, 3), lambda b, i: (b, i, 0)),
          pl.BlockSpec((3, _COUT), lambda b, i: (0, 0)),
          pl.BlockSpec((_COUT, _COUT), lambda b, i: (0, 0)),
          pl.BlockSpec((1, _COUT), lambda b, i: (0, 0)),
      ],
      out_specs=[
          pl.BlockSpec((1, r, _COUT), lambda b, i: (b, i, 0)),
          pl.BlockSpec((2, _COUT), lambda b, i: (0, 0)),
      ],
      out_shape=[
          jax.ShapeDtypeStruct((_B, n1, _COUT), jnp.float32),
          jax.ShapeDtypeStruct((2, _COUT), jnp.float32),
      ],
  )(g, pos, w1p, w11, b11)


def _mlp2_body(g_ref, pos_ref, feat_ref, w2p_ref, w_ref, b_ref, wres_ref,
               ch_ref, res_ref, sums_ref):
  p = pos_ref[0]
  r = p.shape[0]
  c = jnp.zeros((r, _COUT), jnp.float32)
  for d in range(3):
    c = c + p[:, d:d + 1] * w2p_ref[d, :][None, :]
  g = g_ref[0, :, :, :_COUT]
  h = jnp.maximum(g - c[:, None, :], 0.0).reshape(r * _K, _COUT)
  y = jnp.dot(h, w_ref[...], preferred_element_type=jnp.float32) + b_ref[...]
  y = jnp.maximum(y, 0.0)
  acc = jnp.max(y.reshape(r, _K, _COUT), axis=1)
  ch_ref[0] = acc
  res = jnp.dot(feat_ref[0], wres_ref[...], preferred_element_type=jnp.float32)
  res_ref[0] = res
  first = (pl.program_id(0) == 0) & (pl.program_id(1) == 0)

  @pl.when(first)
  def _():
    sums_ref[...] = jnp.zeros_like(sums_ref)

  sums_ref[...] += jnp.concatenate([
      jnp.sum(acc, axis=0, keepdims=True),
      jnp.sum(acc * acc, axis=0, keepdims=True),
      jnp.sum(res, axis=0, keepdims=True),
      jnp.sum(res * res, axis=0, keepdims=True),
  ], axis=0)


def _mlp2(g, pos, feat, w2p, w21, b21, wres, r=256):
  n2 = _N // 4
  grid = (_B, n2 // r)
  return pl.pallas_call(
      _mlp2_body,
      grid=grid,
      in_specs=[
          pl.BlockSpec((1, r, _K, _D), lambda b, i: (b, i, 0, 0)),
          pl.BlockSpec((1, r, 3), lambda b, i: (b, i, 0)),
          pl.BlockSpec((1, r, _CIN), lambda b, i: (b, i, 0)),
          pl.BlockSpec((3, _COUT), lambda b, i: (0, 0)),
          pl.BlockSpec((_COUT, _COUT), lambda b, i: (0, 0)),
          pl.BlockSpec((1, _COUT), lambda b, i: (0, 0)),
          pl.BlockSpec((_CIN, _COUT), lambda b, i: (0, 0)),
      ],
      out_specs=[
          pl.BlockSpec((1, r, _COUT), lambda b, i: (b, i, 0)),
          pl.BlockSpec((1, r, _COUT), lambda b, i: (b, i, 0)),
          pl.BlockSpec((4, _COUT), lambda b, i: (0, 0)),
      ],
      out_shape=[
          jax.ShapeDtypeStruct((_B, n2, _COUT), jnp.float32),
          jax.ShapeDtypeStruct((_B, n2, _COUT), jnp.float32),
          jax.ShapeDtypeStruct((4, _COUT), jnp.float32),
      ],
  )(g, pos, feat, w2p, w21, b21, wres)


def _fin_body(ch_ref, res_ref, sums_ref, g2_ref, be2_ref, gr_ref, br_ref,
              out_ref, *, count):
  m2 = sums_ref[0, :] / count
  v2 = sums_ref[1, :] / count - m2 * m2
  a2 = g2_ref[0, :] * lax.rsqrt(v2 + _EPS)
  c2 = be2_ref[0, :] - m2 * a2
  mr = sums_ref[2, :] / count
  vr = sums_ref[3, :] / count - mr * mr
  ar = gr_ref[0, :] * lax.rsqrt(vr + _EPS)
  cr = br_ref[0, :] - mr * ar
  out_ref[0] = (ch_ref[0] * a2[None, :] + c2[None, :]
                + res_ref[0] * ar[None, :] + cr[None, :])


def _finalize(ch2, res2, sums, g2, be2, gr, br, r=512):
  n2 = _N // 4
  grid = (_B, n2 // r)
  return pl.pallas_call(
      functools.partial(_fin_body, count=float(_B * n2)),
      grid=grid,
      in_specs=[
          pl.BlockSpec((1, r, _COUT), lambda b, i: (b, i, 0)),
          pl.BlockSpec((1, r, _COUT), lambda b, i: (b, i, 0)),
          pl.BlockSpec((4, _COUT), lambda b, i: (0, 0)),
          pl.BlockSpec((1, _COUT), lambda b, i: (0, 0)),
          pl.BlockSpec((1, _COUT), lambda b, i: (0, 0)),
          pl.BlockSpec((1, _COUT), lambda b, i: (0, 0)),
          pl.BlockSpec((1, _COUT), lambda b, i: (0, 0)),
      ],
      out_specs=pl.BlockSpec((1, r, _COUT), lambda b, i: (b, i, 0)),
      out_shape=jax.ShapeDtypeStruct((_B, n2, _COUT), jnp.float32),
  )(ch2, res2, sums, g2, be2, gr, br)


def kernel(position_matrix, channel_matrix, W1_0, b1_0, W1_1, b1_1, W_res,
           W2_0, b2_0, W2_1, b2_1, gamma1, beta1, gamma2, beta2, gamma_r,
           beta_r, n_select_0, n_select_1, n_select_2):
  n0, n1, n2 = _N, _N // 2, _N // 4
  zero = ((jnp.asarray(n_select_0) - n0)
          + (jnp.asarray(n_select_1) - n1)
          + (jnp.asarray(n_select_2) - n2)).astype(position_matrix.dtype)

  pos = position_matrix
  feat = channel_matrix
  w1p, w1f = W1_0[:3], W1_0[3:]
  w2p, w2f = W2_0[:3], W2_0[3:]
  row = lambda v: v.reshape(1, _COUT)

  idx1 = _topk(pos, n0, n1)
  a1 = _build_a1(pos, feat, w1p, w1f, row(b1_0))
  g1 = _gather_rows(a1.reshape(_B * n0, _D), idx1)
  ch1, sums1 = _mlp1(g1, pos, w1p, W1_1, row(b1_1))

  pos1 = pos[:, :n1]
  idx2 = _topk(pos1, n1, n2)
  a2 = _build_a2(pos1, ch1, sums1, w2p, w2f, row(b2_0), row(gamma1),
                 row(beta1))
  g2 = _gather_rows(a2.reshape(_B * n1, _D), idx2)
  ch2_raw, res2, sums2 = _mlp2(g2, pos1, feat[:, :n2], w2p, W2_1, row(b2_1),
                               W_res)
  ch2 = _finalize(ch2_raw, res2, sums2, row(gamma2), row(beta2), row(gamma_r),
                  row(beta_r))
  pos2 = pos[:, :n2] + zero
  return (pos2, ch2)
